# Initial kernel scaffold; baseline (speedup 1.0000x reference)
import jax
import jax.numpy as jnp
from jax.experimental import pallas as pl


def kernel(x_source, neighborhood_indices, neighborhood_values):
    raise NotImplementedError("write your pallas kernel here")


# speedup vs baseline: 3.9989x; 3.9989x over previous
import functools

import jax
import jax.numpy as jnp
from jax import lax
from jax.experimental import pallas as pl
from jax.experimental.pallas import tpu as pltpu
from jax.experimental.pallas import tpu_sc as plsc

N_NODES = 10000
D_FEAT = 128
N_EDGES = 320000

_NC = 2
_NS = 16
_NW = _NC * _NS
_EPT = N_EDGES // _NW
_K = 80
_NCHUNK = _EPT // _K
_N_PAD = 10240
_RPT = _N_PAD // _NS
_ZR = 80


def _sc_body(x_hbm, src_hbm, tgt_hbm, vals_hbm, out_hbm,
             acc_sh, rows_v, src_v, tgt_v, vals_v, stage_v, sem):
    c = lax.axis_index("c")
    s = lax.axis_index("s")
    wid = s * _NC + c

    zeros16 = jnp.zeros((16,), jnp.float32)

    def zbody(i, _):
        stage_v[i, pl.ds(0 * 16, 16)] = zeros16
        for d in range(1, D_FEAT // 16):
            stage_v[i, pl.ds(d * 16, 16)] = zeros16
        return 0

    lax.fori_loop(0, _ZR, zbody, 0)
    r0 = s * _RPT
    for j in range(_RPT // _ZR):
        pltpu.sync_copy(stage_v, acc_sh.at[pl.ds(r0 + j * _ZR, _ZR)])
    plsc.subcore_barrier()

    def chunk_body(ci, _):
        base = wid * _EPT + ci * _K
        pltpu.sync_copy(src_hbm.at[pl.ds(base, _K)], src_v)
        pltpu.sync_copy(tgt_hbm.at[pl.ds(base, _K)], tgt_v)
        pltpu.sync_copy(vals_hbm.at[pl.ds(base, _K)], vals_v)
        pltpu.async_copy(x_hbm.at[src_v], rows_v, sem).wait()

        def scale_body(k, _):
            vs = plsc.load_gather(vals_v, [jnp.full((16,), k, jnp.int32)])
            for d in range(D_FEAT // 16):
                sl = pl.ds(d * 16, 16)
                rows_v[k, sl] = rows_v[k, sl] * vs
            return 0

        lax.fori_loop(0, _K, scale_body, 0)
        pltpu.sync_copy(rows_v, acc_sh.at[tgt_v], add=True)
        return 0

    lax.fori_loop(0, _NCHUNK, chunk_body, 0)
    plsc.subcore_barrier()

    for j in range(_RPT // _ZR):
        rr = r0 + j * _ZR
        pltpu.sync_copy(acc_sh.at[pl.ds(rr, _ZR)], stage_v)
        pltpu.sync_copy(stage_v, out_hbm.at[c].at[pl.ds(rr, _ZR)])


def _tc_add_body(a_ref, b_ref, o_ref):
    o_ref[...] = a_ref[...] + b_ref[...]


def kernel(x_source, neighborhood_indices, neighborhood_values):
    tgt = neighborhood_indices[0]
    src = neighborhood_indices[1]

    mesh = plsc.VectorSubcoreMesh(core_axis_name="c", subcore_axis_name="s")
    partials = pl.kernel(
        _sc_body,
        mesh=mesh,
        compiler_params=pltpu.CompilerParams(needs_layout_passes=False),
        out_type=jax.ShapeDtypeStruct((_NC, _N_PAD, D_FEAT), jnp.float32),
        scratch_types=[
            pltpu.VMEM_SHARED((_N_PAD, D_FEAT), jnp.float32),
            pltpu.VMEM((_K, D_FEAT), jnp.float32),
            pltpu.VMEM((_K,), jnp.int32),
            pltpu.VMEM((_K,), jnp.int32),
            pltpu.VMEM((_K,), jnp.float32),
            pltpu.VMEM((_ZR, D_FEAT), jnp.float32),
            pltpu.SemaphoreType.DMA,
        ],
    )(x_source, src, tgt, neighborhood_values)

    blk = 1000
    out = pl.pallas_call(
        _tc_add_body,
        out_shape=jax.ShapeDtypeStruct((N_NODES, D_FEAT), jnp.float32),
        grid=(N_NODES // blk,),
        in_specs=[
            pl.BlockSpec((blk, D_FEAT), lambda i: (i, 0)),
            pl.BlockSpec((blk, D_FEAT), lambda i: (i, 0)),
        ],
        out_specs=pl.BlockSpec((blk, D_FEAT), lambda i: (i, 0)),
    )(partials[0], partials[1])
    return out


# speedup vs baseline: 9.3324x; 2.3338x over previous
import jax
import jax.numpy as jnp
from jax import lax
from jax.experimental import pallas as pl
from jax.experimental.pallas import tpu as pltpu
from jax.experimental.pallas import tpu_sc as plsc

N_NODES = 10000
D_FEAT = 128
N_EDGES = 320000

_NC = 2
_NS = 16
_NW = _NC * _NS
_EPT = N_EDGES // _NW
_K = 80
_NPASS = 5
_EPP = _EPT // _NPASS
_CPP = _EPP // _K
_N_PAD = 10240
_RPT = _N_PAD // _NS


def _scale_chunk(rows, vals_p, ci):
    def gbody(g, _):
        vv = vals_p[pl.ds(ci * _K + g * 16, 16)]
        for j in range(16):
            vs = jnp.broadcast_to(vv[j], (16,))
            r = g * 16 + j
            for d in range(D_FEAT // 16):
                sl = pl.ds(d * 16, 16)
                rows[r, sl] = rows[r, sl] * vs
        return 0

    lax.fori_loop(0, _K // 16, gbody, 0)


def _sc_body(x_hbm, src_hbm, tgt_hbm, vals_hbm, out_hbm,
             acc_sh, rows_a, rows_b, src_p, tgt_p, vals_p, sem_a, sem_b):
    c = lax.axis_index("c")
    s = lax.axis_index("s")
    wid = s * _NC + c

    zeros16 = jnp.zeros((16,), jnp.float32)

    def zbody(i, _):
        for d in range(D_FEAT // 16):
            rows_a[i, pl.ds(d * 16, 16)] = zeros16
        return 0

    lax.fori_loop(0, _K, zbody, 0)
    r0 = s * _RPT
    for j in range(_RPT // _K):
        pltpu.sync_copy(rows_a, acc_sh.at[pl.ds(r0 + j * _K, _K)])
    plsc.subcore_barrier()

    def gather_start(ci, rows, sem):
        pltpu.async_copy(x_hbm.at[src_p.at[pl.ds(ci * _K, _K)]], rows, sem)

    def gather_wait(ci, rows, sem):
        pltpu.make_async_copy(
            x_hbm.at[src_p.at[pl.ds(ci * _K, _K)]], rows, sem).wait()

    def process(ci, rows):
        _scale_chunk(rows, vals_p, ci)
        for g in range(_K // 16):
            tv = tgt_p[pl.ds(ci * _K + g * 16, 16)]
            pltpu.sync_copy(rows.at[pl.ds(g * 16, 16)], acc_sh.at[tv],
                            add=True)

    for ps in range(_NPASS):
        e0 = wid * _EPT + ps * _EPP
        pltpu.sync_copy(src_hbm.at[pl.ds(e0, _EPP)], src_p)
        pltpu.sync_copy(tgt_hbm.at[pl.ds(e0, _EPP)], tgt_p)
        pltpu.sync_copy(vals_hbm.at[pl.ds(e0, _EPP)], vals_p)

        gather_start(0, rows_a, sem_a)

        def pbody(p, _):
            ci0 = 2 * p
            gather_wait(ci0, rows_a, sem_a)
            gather_start(ci0 + 1, rows_b, sem_b)
            process(ci0, rows_a)
            gather_wait(ci0 + 1, rows_b, sem_b)
            gather_start(ci0 + 2, rows_a, sem_a)
            process(ci0 + 1, rows_b)
            return 0

        lax.fori_loop(0, (_CPP - 1) // 2, pbody, 0)
        gather_wait(_CPP - 1, rows_a, sem_a)
        process(_CPP - 1, rows_a)

    plsc.subcore_barrier()

    for j in range(_RPT // _K):
        rr = r0 + j * _K
        pltpu.sync_copy(acc_sh.at[pl.ds(rr, _K)], rows_a)
        pltpu.sync_copy(rows_a, out_hbm.at[c].at[pl.ds(rr, _K)])


def _tc_add_body(a_ref, b_ref, o_ref):
    o_ref[...] = a_ref[...] + b_ref[...]


def kernel(x_source, neighborhood_indices, neighborhood_values):
    tgt = neighborhood_indices[0]
    src = neighborhood_indices[1]

    mesh = plsc.VectorSubcoreMesh(core_axis_name="c", subcore_axis_name="s")
    partials = pl.kernel(
        _sc_body,
        mesh=mesh,
        compiler_params=pltpu.CompilerParams(needs_layout_passes=False),
        out_type=jax.ShapeDtypeStruct((_NC, _N_PAD, D_FEAT), jnp.float32),
        scratch_types=[
            pltpu.VMEM_SHARED((_N_PAD, D_FEAT), jnp.float32),
            pltpu.VMEM((_K, D_FEAT), jnp.float32),
            pltpu.VMEM((_K, D_FEAT), jnp.float32),
            pltpu.VMEM((_EPP,), jnp.int32),
            pltpu.VMEM((_EPP,), jnp.int32),
            pltpu.VMEM((_EPP,), jnp.float32),
            pltpu.SemaphoreType.DMA,
            pltpu.SemaphoreType.DMA,
        ],
    )(x_source, src, tgt, neighborhood_values)

    blk = 1000
    out = pl.pallas_call(
        _tc_add_body,
        out_shape=jax.ShapeDtypeStruct((N_NODES, D_FEAT), jnp.float32),
        grid=(N_NODES // blk,),
        in_specs=[
            pl.BlockSpec((blk, D_FEAT), lambda i: (i, 0)),
            pl.BlockSpec((blk, D_FEAT), lambda i: (i, 0)),
        ],
        out_specs=pl.BlockSpec((blk, D_FEAT), lambda i: (i, 0)),
    )(partials[0], partials[1])
    return out


# speedup vs baseline: 9.3725x; 1.0043x over previous
import jax
import jax.numpy as jnp
from jax import lax
from jax.experimental import pallas as pl
from jax.experimental.pallas import tpu as pltpu
from jax.experimental.pallas import tpu_sc as plsc

N_NODES = 10000
D_FEAT = 128
N_EDGES = 320000

_NC = 2
_NS = 16
_NW = _NC * _NS
_EPT = N_EDGES // _NW
_K = 80
_NPASS = 5
_EPP = _EPT // _NPASS
_CPP = _EPP // _K
_N_PAD = 10240
_RPT = _N_PAD // _NS


def _scale_chunk(rows, vals_p, ci):
    def gbody(g, _):
        vv = vals_p[pl.ds(ci * _K + g * 16, 16)]
        for j in range(16):
            vs = jnp.broadcast_to(vv[j], (16,))
            r = g * 16 + j
            for d in range(D_FEAT // 16):
                sl = pl.ds(d * 16, 16)
                rows[r, sl] = rows[r, sl] * vs
        return 0

    lax.fori_loop(0, _K // 16, gbody, 0)


def _sc_body(x_hbm, src_hbm, tgt_hbm, vals_hbm, out_hbm,
             acc_sh, rows_a, rows_b, src_p, tgt_p, vals_p,
             sem_a, sem_b, sem_sa, sem_sb):
    c = lax.axis_index("c")
    s = lax.axis_index("s")
    wid = s * _NC + c

    zeros16 = jnp.zeros((16,), jnp.float32)

    def zbody(i, _):
        for d in range(D_FEAT // 16):
            rows_a[i, pl.ds(d * 16, 16)] = zeros16
        return 0

    lax.fori_loop(0, _K, zbody, 0)
    r0 = s * _RPT
    for j in range(_RPT // _K):
        pltpu.sync_copy(rows_a, acc_sh.at[pl.ds(r0 + j * _K, _K)])
    plsc.subcore_barrier()

    def gather_start(ci, rows, sem):
        pltpu.async_copy(x_hbm.at[src_p.at[pl.ds(ci * _K, _K)]], rows, sem)

    def gather_wait(ci, rows, sem):
        pltpu.make_async_copy(
            x_hbm.at[src_p.at[pl.ds(ci * _K, _K)]], rows, sem).wait()

    def scat_fire(ci, rows, sem):
        for g in range(_K // 16):
            tv = tgt_p[pl.ds(ci * _K + g * 16, 16)]
            pltpu.async_copy(rows.at[pl.ds(g * 16, 16)], acc_sh.at[tv], sem,
                             add=True)

    def scat_drain(ci, rows, sem):
        for g in range(_K // 16):
            tv = tgt_p[pl.ds(ci * _K + g * 16, 16)]
            pltpu.make_async_copy(rows.at[pl.ds(g * 16, 16)], acc_sh.at[tv],
                                  sem).wait()

    for ps in range(_NPASS):
        e0 = wid * _EPT + ps * _EPP
        pltpu.sync_copy(src_hbm.at[pl.ds(e0, _EPP)], src_p)
        pltpu.sync_copy(tgt_hbm.at[pl.ds(e0, _EPP)], tgt_p)
        pltpu.sync_copy(vals_hbm.at[pl.ds(e0, _EPP)], vals_p)

        gather_start(0, rows_a, sem_a)

        def pbody(p, _):
            ci0 = 2 * p
            gather_wait(ci0, rows_a, sem_a)

            @pl.when(p > 0)
            def _():
                scat_drain(ci0 - 1, rows_b, sem_sb)

            gather_start(ci0 + 1, rows_b, sem_b)
            _scale_chunk(rows_a, vals_p, ci0)
            scat_fire(ci0, rows_a, sem_sa)
            gather_wait(ci0 + 1, rows_b, sem_b)
            scat_drain(ci0, rows_a, sem_sa)
            gather_start(ci0 + 2, rows_a, sem_a)
            _scale_chunk(rows_b, vals_p, ci0 + 1)
            scat_fire(ci0 + 1, rows_b, sem_sb)
            return 0

        lax.fori_loop(0, (_CPP - 1) // 2, pbody, 0)
        gather_wait(_CPP - 1, rows_a, sem_a)
        scat_drain(_CPP - 2, rows_b, sem_sb)
        _scale_chunk(rows_a, vals_p, _CPP - 1)
        scat_fire(_CPP - 1, rows_a, sem_sa)
        scat_drain(_CPP - 1, rows_a, sem_sa)

    plsc.subcore_barrier()

    for j in range(_RPT // _K):
        rr = r0 + j * _K
        pltpu.async_copy(acc_sh.at[pl.ds(rr, _K)],
                         out_hbm.at[c].at[pl.ds(rr, _K)], sem_a)
    for j in range(_RPT // _K):
        rr = r0 + j * _K
        pltpu.make_async_copy(acc_sh.at[pl.ds(rr, _K)],
                              out_hbm.at[c].at[pl.ds(rr, _K)], sem_a).wait()


def _tc_add_body(a_ref, b_ref, o_ref):
    o_ref[...] = a_ref[...] + b_ref[...]


def kernel(x_source, neighborhood_indices, neighborhood_values):
    tgt = neighborhood_indices[0]
    src = neighborhood_indices[1]

    mesh = plsc.VectorSubcoreMesh(core_axis_name="c", subcore_axis_name="s")
    partials = pl.kernel(
        _sc_body,
        mesh=mesh,
        compiler_params=pltpu.CompilerParams(needs_layout_passes=False),
        out_type=jax.ShapeDtypeStruct((_NC, _N_PAD, D_FEAT), jnp.float32),
        scratch_types=[
            pltpu.VMEM_SHARED((_N_PAD, D_FEAT), jnp.float32),
            pltpu.VMEM((_K, D_FEAT), jnp.float32),
            pltpu.VMEM((_K, D_FEAT), jnp.float32),
            pltpu.VMEM((_EPP,), jnp.int32),
            pltpu.VMEM((_EPP,), jnp.int32),
            pltpu.VMEM((_EPP,), jnp.float32),
            pltpu.SemaphoreType.DMA,
            pltpu.SemaphoreType.DMA,
            pltpu.SemaphoreType.DMA,
            pltpu.SemaphoreType.DMA,
        ],
    )(x_source, src, tgt, neighborhood_values)

    blk = 1000
    out = pl.pallas_call(
        _tc_add_body,
        out_shape=jax.ShapeDtypeStruct((N_NODES, D_FEAT), jnp.float32),
        grid=(N_NODES // blk,),
        in_specs=[
            pl.BlockSpec((blk, D_FEAT), lambda i: (i, 0)),
            pl.BlockSpec((blk, D_FEAT), lambda i: (i, 0)),
        ],
        out_specs=pl.BlockSpec((blk, D_FEAT), lambda i: (i, 0)),
    )(partials[0], partials[1])
    return out
